# baseline (speedup 1.0000x reference)
import functools

import jax
import jax.numpy as jnp
from jax import lax
from jax.experimental import pallas as pl
from jax.experimental.pallas import tpu as pltpu
from jax.experimental.pallas import tpu_sc as plsc

D_MODEL = 64
SCALE = 8.0
C = 128


def kernel(x, table):
    out_shape = (*x.shape, D_MODEL)
    B = x.size
    V = table.shape[0]

    info = plsc.get_sparse_core_info()
    NC, NS = info.num_cores, info.num_subcores
    NW = NC * NS
    BPW = B // NW
    NCH = BPW // C
    assert BPW * NW == B and NCH * C == BPW and NCH % 2 == 0

    x_rows = jnp.reshape(x.astype(jnp.int32), (NW * NCH, C))
    table2 = jnp.reshape(table, (V // 2, 2 * D_MODEL))

    mesh = plsc.VectorSubcoreMesh(core_axis_name="c", subcore_axis_name="s")

    @functools.partial(
        pl.kernel,
        mesh=mesh,
        out_type=jax.ShapeDtypeStruct((B, D_MODEL), jnp.float32),
        compiler_params=pltpu.CompilerParams(needs_layout_passes=False),
        scratch_types=[
            pltpu.VMEM((NCH, C), jnp.int32),
            pltpu.VMEM((2, C), jnp.int32),
            pltpu.VMEM((2, C), jnp.int32),
            pltpu.VMEM((2, C, 2 * D_MODEL), jnp.float32),
            pltpu.VMEM((C, D_MODEL), jnp.float32),
            pltpu.SemaphoreType.DMA,
            pltpu.SemaphoreType.DMA,
        ],
    )
    def emb(x_hbm, table_hbm, out_hbm, idx_all, kbuf, hbuf, pairs, outb,
            sem0, sem1):
        wid = lax.axis_index("c") * NS + lax.axis_index("s")
        pltpu.sync_copy(x_hbm.at[pl.ds(wid * NCH, NCH)], idx_all)

        sems = (sem0, sem1)

        def prep_indices(n, b):
            def body(m, _):
                sl = pl.ds(m * 16, 16)
                v = idx_all[n, sl]
                kbuf[b, sl] = lax.shift_right_logical(v, 1)
                hbuf[b, sl] = lax.bitwise_and(v, 1)
                return 0
            lax.fori_loop(0, C // 16, body, 0)

        def gather_start(b):
            pltpu.make_async_copy(
                table_hbm.at[kbuf.at[b]], pairs.at[b], sems[b]
            ).start()

        def gather_wait(b):
            pltpu.make_async_copy(
                table_hbm.at[kbuf.at[b]], pairs.at[b], sems[b]
            ).wait()

        prep_indices(0, 0)
        gather_start(0)

        out_base = wid * BPW

        def outer(i, _):
            n0 = i * 2
            for b in range(2):
                n = n0 + b
                nxt = n + 1

                @pl.when(nxt < NCH)
                def _():
                    prep_indices(nxt, 1 - b)
                    gather_start(1 - b)

                gather_wait(b)

                def select_scale(r2, _):
                    for rr in range(2):
                        r = r2 * 2 + rr
                        hv = plsc.load_gather(
                            hbuf, [jnp.full((16,), b, jnp.int32),
                                   jnp.full((16,), r, jnp.int32)]
                        )
                        pick_hi = hv > 0
                        for m in range(D_MODEL // 16):
                            lo = pairs[b, r, pl.ds(m * 16, 16)]
                            hi = pairs[b, r, pl.ds(D_MODEL + m * 16, 16)]
                            outb[r, pl.ds(m * 16, 16)] = (
                                jnp.where(pick_hi, hi, lo) * SCALE
                            )
                    return 0

                lax.fori_loop(0, C // 2, select_scale, 0)

                pltpu.sync_copy(outb, out_hbm.at[pl.ds(out_base + n * C, C)])
            return 0

        lax.fori_loop(0, NCH // 2, outer, 0)

    out = emb(x_rows, table2)
    return out.reshape(out_shape)
